# baseline (speedup 1.0000x reference)
import functools

import jax
import jax.numpy as jnp
import numpy as np
from jax import lax
from jax.experimental import pallas as pl
from jax.experimental.pallas import tpu as pltpu
from jax.experimental.pallas import tpu_sc as plsc

N = 10000
N2 = 5000
E0 = 320000
E1 = 160000
EU = 30000
C = 128

NC, NS, NW = 2, 16, 32
K = 128

NACC_N = 10112
NACC_N2 = 5120
E0P = ((E0 + NW * K - 1) // (NW * K)) * (NW * K)
E1P = ((E1 + NW * K - 1) // (NW * K)) * (NW * K)
EUP = ((EU + NW * K - 1) // (NW * K)) * (NW * K)
GPAD = 5120

_HI = jax.lax.Precision.HIGHEST

_P1 = np.zeros((C, C), np.float32)
_P2 = np.zeros((C, C), np.float32)
for _j in range(C):
    _P1[_j, _j // 2] = 1.0
    _P2[_j, C // 2 + _j // 2] = 1.0

_MESH = plsc.VectorSubcoreMesh(core_axis_name="c", subcore_axis_name="s",
                               num_cores=NC, num_subcores=NS)


def _seg_call(ta, tq, src, dst, z128, n_acc, *, name, cw=0):
    two = tq is not None
    e_pad = src.shape[0]
    e_per_w = e_pad // NW
    n_chunks = e_per_w // K
    rows_sub = n_acc // NS

    out_type = [jax.ShapeDtypeStruct((NC, n_acc, C), jnp.float32)]
    if cw:
        out_type.append(jax.ShapeDtypeStruct((NC, n_acc, cw), jnp.float32))

    scratch = [pltpu.VMEM((K,), jnp.int32),
               pltpu.VMEM((K,), jnp.int32),
               pltpu.VMEM((K, C), jnp.float32)]
    if two:
        scratch.append(pltpu.VMEM((K, C), jnp.float32))
    if cw:
        scratch.append(pltpu.VMEM((K, cw), jnp.float32))
    scratch.append(pltpu.VMEM_SHARED((n_acc, C), jnp.float32))
    if cw:
        scratch.append(pltpu.VMEM_SHARED((n_acc, cw), jnp.float32))
    scratch.append(pltpu.SemaphoreType.DMA)
    if two:
        scratch.append(pltpu.SemaphoreType.DMA)

    def body(*refs):
        it = iter(refs)
        ta_ref = next(it)
        tq_ref = next(it) if two else None
        src_ref = next(it)
        dst_ref = next(it)
        z128_ref = next(it)
        out_sum = next(it)
        out_cnt = next(it) if cw else None
        src_v = next(it)
        dst_v = next(it)
        arows = next(it)
        qrows = next(it) if two else None
        ones_v = next(it) if cw else None
        acc = next(it)
        cacc = next(it) if cw else None
        sem_a = next(it)
        sem_q = next(it) if two else None

        cid = lax.axis_index("c")
        sid = lax.axis_index("s")
        wid = sid * NC + cid
        r0 = sid * rows_sub

        pltpu.sync_copy(z128_ref.at[pl.ds(r0, rows_sub)],
                        acc.at[pl.ds(r0, rows_sub)])
        if cw:
            @pl.loop(0, K)
            def _(r):
                for j in range(cw // 16):
                    ones_v[r, pl.ds(j * 16, 16)] = jnp.full((16,), 0.0,
                                                            jnp.float32)

            nfull, rem = rows_sub // K, rows_sub % K
            for k in range(nfull):
                pltpu.sync_copy(ones_v, cacc.at[pl.ds(r0 + k * K, K)])
            if rem:
                pltpu.sync_copy(ones_v.at[pl.ds(0, rem)],
                                cacc.at[pl.ds(r0 + nfull * K, rem)])

            @pl.loop(0, K)
            def _(r):
                for j in range(cw // 16):
                    ones_v[r, pl.ds(j * 16, 16)] = jnp.full((16,), 1.0,
                                                            jnp.float32)

        plsc.subcore_barrier()

        base = wid * e_per_w

        @pl.loop(0, n_chunks)
        def _(ci):
            off = base + ci * K
            pltpu.sync_copy(src_ref.at[pl.ds(off, K)], src_v)
            pltpu.sync_copy(dst_ref.at[pl.ds(off, K)], dst_v)
            cp_a = pltpu.async_copy(ta_ref.at[src_v], arows, sem_a)
            if two:
                cp_q = pltpu.async_copy(tq_ref.at[dst_v], qrows, sem_q)
            if cw:
                pltpu.sync_copy(ones_v, cacc.at[dst_v], add=True)
            cp_a.wait()
            if two:
                cp_q.wait()

                @pl.loop(0, K)
                def _(r):
                    for j in range(C // 16):
                        sl = pl.ds(j * 16, 16)
                        arows[r, sl] = jnp.maximum(arows[r, sl] + qrows[r, sl],
                                                   0.0)

            pltpu.sync_copy(arows, acc.at[dst_v], add=True)

        plsc.subcore_barrier()
        pltpu.sync_copy(acc.at[pl.ds(r0, rows_sub)],
                        out_sum.at[cid, pl.ds(r0, rows_sub)])
        if cw:
            pltpu.sync_copy(cacc.at[pl.ds(r0, rows_sub)],
                            out_cnt.at[cid, pl.ds(r0, rows_sub)])

    ins = [ta] + ([tq] if two else []) + [src, dst, z128]
    fn = pl.kernel(body, out_type=out_type if cw else out_type[0], mesh=_MESH,
                   scratch_types=scratch, name=name)
    return fn(*ins)


def _hist3_call(dsta, dstb, dstc, *, name):
    counts = [(dsta, NACC_N), (dstb, NACC_N2), (dstc, NACC_N)]
    rows_sub = NACC_N // NS
    rows_sub2 = NACC_N2 // NS

    def body(dsta_ref, dstb_ref, dstc_ref, out_a, out_b, out_c,
             dst_v, ones_v, zero_v, cnt_acc):
        cid = lax.axis_index("c")
        sid = lax.axis_index("s")
        wid = sid * NC + cid

        @pl.loop(0, K)
        def _(r):
            for j in range(C // 16):
                ones_v[r, pl.ds(j * 16, 16)] = jnp.full((16,), 1.0,
                                                        jnp.float32)
                zero_v[r, pl.ds(j * 16, 16)] = jnp.full((16,), 0.0,
                                                        jnp.float32)

        for (dst_ref, out_ref, n_acc) in ((dsta_ref, out_a, NACC_N),
                                          (dstb_ref, out_b, NACC_N2),
                                          (dstc_ref, out_c, NACC_N)):
            rs = n_acc // NS
            r0 = sid * rs
            nfull, rem = rs // K, rs % K
            for k in range(nfull):
                pltpu.sync_copy(zero_v, cnt_acc.at[pl.ds(r0 + k * K, K)])
            if rem:
                pltpu.sync_copy(zero_v.at[pl.ds(0, rem)],
                                cnt_acc.at[pl.ds(r0 + nfull * K, rem)])
            plsc.subcore_barrier()

            e_per_w = dst_ref.shape[0] // NW
            base = wid * e_per_w

            @pl.loop(0, e_per_w // K)
            def _(ci):
                pltpu.sync_copy(dst_ref.at[pl.ds(base + ci * K, K)], dst_v)
                pltpu.sync_copy(ones_v, cnt_acc.at[dst_v], add=True)

            plsc.subcore_barrier()
            pltpu.sync_copy(cnt_acc.at[pl.ds(r0, rs)],
                            out_ref.at[cid, pl.ds(r0, rs)])
            plsc.subcore_barrier()

    fn = pl.kernel(
        body,
        out_type=[jax.ShapeDtypeStruct((NC, NACC_N, C), jnp.float32),
                  jax.ShapeDtypeStruct((NC, NACC_N2, C), jnp.float32),
                  jax.ShapeDtypeStruct((NC, NACC_N, C), jnp.float32)],
        mesh=_MESH,
        scratch_types=[pltpu.VMEM((K,), jnp.int32),
                       pltpu.VMEM((K, C), jnp.float32),
                       pltpu.VMEM((K, C), jnp.float32),
                       pltpu.VMEM_SHARED((NACC_N, C), jnp.float32)],
        name=name)
    return fn(dsta, dstb, dstc)


def _gather_rows(table, idx, *, name):
    g = idx.shape[0]
    per_w = g // NW
    ck = next(c for c in range(min(per_w, K), 0, -8) if per_w % c == 0)
    n_chunks = per_w // ck

    def body(tab_ref, idx_ref, out_ref, idx_v, rows_v, sem):
        cid = lax.axis_index("c")
        sid = lax.axis_index("s")
        wid = sid * NC + cid
        base = wid * per_w

        @pl.loop(0, n_chunks)
        def _(ci):
            off = base + ci * ck
            pltpu.sync_copy(idx_ref.at[pl.ds(off, ck)], idx_v)
            pltpu.async_copy(tab_ref.at[idx_v], rows_v, sem).wait()
            pltpu.sync_copy(rows_v, out_ref.at[pl.ds(off, ck)])

    fn = pl.kernel(body,
                   out_type=jax.ShapeDtypeStruct((g, C), jnp.float32),
                   mesh=_MESH,
                   scratch_types=[pltpu.VMEM((ck,), jnp.int32),
                                  pltpu.VMEM((ck, C), jnp.float32),
                                  pltpu.SemaphoreType.DMA],
                   name=name)
    return fn(table, idx)


def _tc(fn, out_shape, *args, name):
    return pl.pallas_call(fn, out_shape=out_shape, name=name)(*args)


def _tc1(bxyz, feat, Wd, Wp, bd):
    def body(bxyz_ref, feat_ref, wd_ref, wp_ref, bd_ref, a_ref, q_ref):
        q = (bxyz_ref[:, 1:2] * wp_ref[0:1, :]
             + bxyz_ref[:, 2:3] * wp_ref[1:2, :]
             + bxyz_ref[:, 3:4] * wp_ref[2:3, :])
        a = jnp.dot(feat_ref[...], wd_ref[...], precision=_HI) + bd_ref[...] - q
        a_ref[...] = a
        q_ref[...] = q

    return _tc(body, [jax.ShapeDtypeStruct((N, C), jnp.float32),
                      jax.ShapeDtypeStruct((N, C), jnp.float32)],
               bxyz, feat, Wd, Wp, bd, name="tc1_aq")


def _combine(sums_ref, cnt_ref, n):
    s = sums_ref[0, :n, :] + sums_ref[1, :n, :]
    d = cnt_ref[0, :n, 0:1] + cnt_ref[1, :n, 0:1]
    return s / jnp.maximum(d, 1.0), d


def _tc2(hs, hc):
    def body(hs_ref, hc_ref, h_ref):
        h, _ = _combine(hs_ref, hc_ref, N)
        h_ref[...] = h

    return _tc(body, jax.ShapeDtypeStruct((N, C), jnp.float32), hs, hc,
               name="tc2_h")


def _tc3(S0, c2, F0p, Wf, Wfs, bf):
    def body(s_ref, c_ref, f0_ref, wf_ref, wfs_ref, bf_ref, f1_ref):
        A0, _ = _combine(s_ref, c_ref, N2)
        f0 = f0_ref[:N2, :]
        f1_ref[...] = jax.nn.relu(jnp.dot(A0, wf_ref[...], precision=_HI)
                                  + jnp.dot(f0, wfs_ref[...], precision=_HI)
                                  + bf_ref[...])

    return _tc(body, jax.ShapeDtypeStruct((N2, C), jnp.float32),
               S0, c2, F0p, Wf, Wfs, bf, name="tc3_flat")


def _tc4(S1, c2, Ws1, bs1):
    def body(s_ref, c_ref, w_ref, b_ref, a1_ref, s1_ref):
        A1, _ = _combine(s_ref, c_ref, N2)
        a1_ref[...] = A1
        s1_ref[...] = jax.nn.relu(jnp.dot(A1, w_ref[...], precision=_HI)
                                  + b_ref[...])

    return _tc(body, [jax.ShapeDtypeStruct((N2, C), jnp.float32),
                      jax.ShapeDtypeStruct((N2, C), jnp.float32)],
               S1, c2, Ws1, bs1, name="tc4_skip1")


def _tc5(S2, c2, F1, Ws2, bs2):
    def body(s_ref, c_ref, f1_ref, w_ref, b_ref, skip_ref):
        A2, _ = _combine(s_ref, c_ref, N2)
        s2 = jnp.dot(A2, w_ref[...], precision=_HI) + b_ref[...]
        skip_ref[...] = jax.nn.relu(s2 + f1_ref[...])

    return _tc(body, jax.ShapeDtypeStruct((N2, C), jnp.float32),
               S2, c2, F1, Ws2, bs2, name="tc5_skip2")


def _tc6(S3, c2, A1, F1, skip, Wm1, Wm2, bm, P1, P2, Wu):
    def body(s_ref, c_ref, a1_ref, f1_ref, sk_ref, wm1_ref, wm2_ref, bm_ref,
             p1_ref, p2_ref, wu_ref, g_ref):
        A3, _ = _combine(s_ref, c_ref, N2)
        merged = jax.nn.relu(jnp.dot(a1_ref[...], wm1_ref[...], precision=_HI)
                             + jnp.dot(A3, wm2_ref[...], precision=_HI)
                             + bm_ref[...])
        fr2 = (merged
               + jnp.dot(f1_ref[...], p1_ref[...], precision=_HI)
               + jnp.dot(sk_ref[...], p2_ref[...], precision=_HI))
        g_ref[...] = jnp.dot(fr2, wu_ref[...], precision=_HI)

    return _tc(body, jax.ShapeDtypeStruct((N2, C), jnp.float32),
               S3, c2, A1, F1, skip, Wm1, Wm2, bm, P1, P2, Wu,
               name="tc6_merge")


def _tc7(G, cu, bu):
    def body(g_ref, c_ref, b_ref, out_ref):
        avg, d = _combine(g_ref, c_ref, N)
        gate = jnp.where(d > 0.0, 1.0, 0.0)
        out_ref[...] = jax.nn.relu(avg + gate * b_ref[...])

    return _tc(body, jax.ShapeDtypeStruct((N, C), jnp.float32), G, cu, bu,
               name="tc7_up")


def kernel(point_bxyz, point_feat, edge_index, sample_idx, edge_index_down,
           up_src, up_dst, W_down, W_pos, b_down, W_flat, W_flat_self, b_flat,
           W_skip1, b_skip1, W_skip2, b_skip2, W_merge, b_merge, W_up, b_up):
    f32 = jnp.float32
    src0 = jnp.pad(edge_index[0], (0, E0P - E0))
    dst0 = jnp.pad(edge_index[1], (0, E0P - E0), constant_values=N)
    src2 = jnp.pad(edge_index_down[0], (0, E1P - E1))
    dst2 = jnp.pad(edge_index_down[1], (0, E1P - E1), constant_values=N2)
    usrc = jnp.pad(up_src, (0, EUP - EU))
    udst = jnp.pad(up_dst, (0, EUP - EU), constant_values=N)
    sidx = jnp.pad(sample_idx, (0, GPAD - N2))

    zN128 = jnp.zeros((NACC_N, C), f32)
    z2128 = jnp.zeros((NACC_N2, C), f32)

    bd = b_down.reshape(1, C)
    bf = b_flat.reshape(1, C)
    bs1 = b_skip1.reshape(1, C)
    bs2 = b_skip2.reshape(1, C)
    bm = b_merge.reshape(1, C)
    bu = b_up.reshape(1, C)
    Wm1, Wm2 = W_merge[:C], W_merge[C:]
    P1 = jnp.asarray(_P1)
    P2 = jnp.asarray(_P2)

    hc, c2, cu = _hist3_call(dst0, dst2, udst, name="sc_deg")
    a, q = _tc1(point_bxyz, point_feat, W_down, W_pos, bd)
    qp = jnp.pad(q, ((0, NACC_N - N), (0, 0)))
    hs = _seg_call(a, qp, src0, dst0, zN128, NACC_N, name="sc_down")
    h = _tc2(hs, hc)
    F0p = _gather_rows(h, sidx, name="sc_sample")
    S0 = _seg_call(F0p, None, src2, dst2, z2128, NACC_N2, name="sc_seg0")
    F1 = _tc3(S0, c2, F0p, W_flat, W_flat_self, bf)
    S1 = _seg_call(F1, None, src2, dst2, z2128, NACC_N2, name="sc_seg1")
    A1, s1 = _tc4(S1, c2, W_skip1, bs1)
    S2 = _seg_call(s1, None, src2, dst2, z2128, NACC_N2, name="sc_seg2")
    skip = _tc5(S2, c2, F1, W_skip2, bs2)
    S3 = _seg_call(skip, None, src2, dst2, z2128, NACC_N2, name="sc_seg3")
    g = _tc6(S3, c2, A1, F1, skip, Wm1, Wm2, bm, P1, P2, W_up)
    G = _seg_call(g, None, usrc, udst, zN128, NACC_N, name="sc_up")
    return _tc7(G, cu, bu)


# speedup vs baseline: 1.0340x; 1.0340x over previous
import functools

import jax
import jax.numpy as jnp
import numpy as np
from jax import lax
from jax.experimental import pallas as pl
from jax.experimental.pallas import tpu as pltpu
from jax.experimental.pallas import tpu_sc as plsc

N = 10000
N2 = 5000
E0 = 320000
E1 = 160000
EU = 30000
C = 128

NC, NS, NW = 2, 16, 32
K = 128

NACC_N = 10112
NACC_N2 = 5120
E0P = ((E0 + NW * K - 1) // (NW * K)) * (NW * K)
E1P = ((E1 + NW * K - 1) // (NW * K)) * (NW * K)
EUP = ((EU + NW * K - 1) // (NW * K)) * (NW * K)
GPAD = 5120

_HI = jax.lax.Precision.HIGHEST

_P1 = np.zeros((C, C), np.float32)
_P2 = np.zeros((C, C), np.float32)
for _j in range(C):
    _P1[_j, _j // 2] = 1.0
    _P2[_j, C // 2 + _j // 2] = 1.0

_MESH = plsc.VectorSubcoreMesh(core_axis_name="c", subcore_axis_name="s",
                               num_cores=NC, num_subcores=NS)


def _seg_call(ta, tq, src, dst, n_acc, *, name):
    two = tq is not None
    e_pad = src.shape[0]
    e_per_w = e_pad // NW
    n_chunks = e_per_w // K
    rows_sub = n_acc // NS

    out_type = jax.ShapeDtypeStruct((NC, n_acc, C), jnp.float32)

    scratch = [pltpu.VMEM((K,), jnp.int32),
               pltpu.VMEM((K,), jnp.int32),
               pltpu.VMEM((K, C), jnp.float32)]
    if two:
        scratch.append(pltpu.VMEM((K, C), jnp.float32))
    scratch.append(pltpu.VMEM_SHARED((n_acc, C), jnp.float32))
    scratch.append(pltpu.SemaphoreType.DMA)
    if two:
        scratch.append(pltpu.SemaphoreType.DMA)

    def body(*refs):
        it = iter(refs)
        ta_ref = next(it)
        tq_ref = next(it) if two else None
        src_ref = next(it)
        dst_ref = next(it)
        out_sum = next(it)
        src_v = next(it)
        dst_v = next(it)
        arows = next(it)
        qrows = next(it) if two else None
        acc = next(it)
        sem_a = next(it)
        sem_q = next(it) if two else None

        cid = lax.axis_index("c")
        sid = lax.axis_index("s")
        wid = sid * NC + cid
        r0 = sid * rows_sub

        @pl.loop(0, K)
        def _(r):
            for j in range(C // 16):
                arows[r, pl.ds(j * 16, 16)] = jnp.full((16,), 0.0,
                                                       jnp.float32)

        nfull, rem = rows_sub // K, rows_sub % K
        for k in range(nfull):
            pltpu.sync_copy(arows, acc.at[pl.ds(r0 + k * K, K)])
        if rem:
            pltpu.sync_copy(arows.at[pl.ds(0, rem)],
                            acc.at[pl.ds(r0 + nfull * K, rem)])
        plsc.subcore_barrier()

        base = wid * e_per_w

        @pl.loop(0, n_chunks)
        def _(ci):
            off = base + ci * K
            pltpu.sync_copy(src_ref.at[pl.ds(off, K)], src_v)
            pltpu.sync_copy(dst_ref.at[pl.ds(off, K)], dst_v)
            cp_a = pltpu.async_copy(ta_ref.at[src_v], arows, sem_a)
            if two:
                cp_q = pltpu.async_copy(tq_ref.at[dst_v], qrows, sem_q)
            cp_a.wait()
            if two:
                cp_q.wait()

                @pl.loop(0, K)
                def _(r):
                    for j in range(C // 16):
                        sl = pl.ds(j * 16, 16)
                        arows[r, sl] = jnp.maximum(arows[r, sl] + qrows[r, sl],
                                                   0.0)

            pltpu.sync_copy(arows, acc.at[dst_v], add=True)

        plsc.subcore_barrier()
        pltpu.sync_copy(acc.at[pl.ds(r0, rows_sub)],
                        out_sum.at[cid, pl.ds(r0, rows_sub)])

    ins = [ta] + ([tq] if two else []) + [src, dst]
    fn = pl.kernel(body, out_type=out_type, mesh=_MESH, scratch_types=scratch,
                   name=name)
    return fn(*ins)


def _hist_call(dst, n_acc, *, name):
    e_pad = dst.shape[0]
    e_per_w = e_pad // NW
    n_chunks = e_per_w // K
    rows_sub = n_acc // NS

    def body(dst_ref, out_cnt, dst_v, ones_v, cnt_acc):
        cid = lax.axis_index("c")
        sid = lax.axis_index("s")
        wid = sid * NC + cid
        r0 = sid * rows_sub

        @pl.loop(0, K)
        def _(r):
            for j in range(C // 16):
                ones_v[r, pl.ds(j * 16, 16)] = jnp.full((16,), 0.0,
                                                        jnp.float32)

        nfull, rem = rows_sub // K, rows_sub % K
        for k in range(nfull):
            pltpu.sync_copy(ones_v, cnt_acc.at[pl.ds(r0 + k * K, K)])
        if rem:
            pltpu.sync_copy(ones_v.at[pl.ds(0, rem)],
                            cnt_acc.at[pl.ds(r0 + nfull * K, rem)])

        @pl.loop(0, K)
        def _(r):
            for j in range(C // 16):
                ones_v[r, pl.ds(j * 16, 16)] = jnp.full((16,), 1.0,
                                                        jnp.float32)

        plsc.subcore_barrier()
        base = wid * e_per_w

        @pl.loop(0, n_chunks)
        def _(ci):
            off = base + ci * K
            pltpu.sync_copy(dst_ref.at[pl.ds(off, K)], dst_v)
            pltpu.sync_copy(ones_v, cnt_acc.at[dst_v], add=True)

        plsc.subcore_barrier()
        pltpu.sync_copy(cnt_acc.at[pl.ds(r0, rows_sub)],
                        out_cnt.at[cid, pl.ds(r0, rows_sub)])

    fn = pl.kernel(body,
                   out_type=jax.ShapeDtypeStruct((NC, n_acc, C), jnp.float32),
                   mesh=_MESH,
                   scratch_types=[pltpu.VMEM((K,), jnp.int32),
                                  pltpu.VMEM((K, C), jnp.float32),
                                  pltpu.VMEM_SHARED((n_acc, C), jnp.float32)],
                   name=name)
    return fn(dst)


def _gather_rows(table, idx, *, name):
    g = idx.shape[0]
    per_w = g // NW
    ck = next(c for c in range(min(per_w, K), 0, -8) if per_w % c == 0)
    n_chunks = per_w // ck

    def body(tab_ref, idx_ref, out_ref, idx_v, rows_v, sem):
        cid = lax.axis_index("c")
        sid = lax.axis_index("s")
        wid = sid * NC + cid
        base = wid * per_w

        @pl.loop(0, n_chunks)
        def _(ci):
            off = base + ci * ck
            pltpu.sync_copy(idx_ref.at[pl.ds(off, ck)], idx_v)
            pltpu.async_copy(tab_ref.at[idx_v], rows_v, sem).wait()
            pltpu.sync_copy(rows_v, out_ref.at[pl.ds(off, ck)])

    fn = pl.kernel(body,
                   out_type=jax.ShapeDtypeStruct((g, C), jnp.float32),
                   mesh=_MESH,
                   scratch_types=[pltpu.VMEM((ck,), jnp.int32),
                                  pltpu.VMEM((ck, C), jnp.float32),
                                  pltpu.SemaphoreType.DMA],
                   name=name)
    return fn(table, idx)


def _tc(fn, out_shape, *args, name):
    return pl.pallas_call(fn, out_shape=out_shape, name=name)(*args)


def _tc1(bxyz, feat, Wd, Wp, bd):
    def body(bxyz_ref, feat_ref, wd_ref, wp_ref, bd_ref, a_ref, q_ref):
        q = (bxyz_ref[:, 1:2] * wp_ref[0:1, :]
             + bxyz_ref[:, 2:3] * wp_ref[1:2, :]
             + bxyz_ref[:, 3:4] * wp_ref[2:3, :])
        a = jnp.dot(feat_ref[...], wd_ref[...], precision=_HI) + bd_ref[...] - q
        a_ref[...] = a
        q_ref[...] = q

    return _tc(body, [jax.ShapeDtypeStruct((N, C), jnp.float32),
                      jax.ShapeDtypeStruct((N, C), jnp.float32)],
               bxyz, feat, Wd, Wp, bd, name="tc1_aq")


def _combine(sums_ref, cnt_ref, n):
    s = sums_ref[0, :n, :] + sums_ref[1, :n, :]
    d = cnt_ref[0, :n, 0:1] + cnt_ref[1, :n, 0:1]
    return s / jnp.maximum(d, 1.0), d


def _tc2(hs, hc):
    def body(hs_ref, hc_ref, h_ref):
        h, _ = _combine(hs_ref, hc_ref, N)
        h_ref[...] = h

    return _tc(body, jax.ShapeDtypeStruct((N, C), jnp.float32), hs, hc,
               name="tc2_h")


def _tc3(S0, c2, F0p, Wf, Wfs, bf):
    def body(s_ref, c_ref, f0_ref, wf_ref, wfs_ref, bf_ref, f1_ref):
        A0, _ = _combine(s_ref, c_ref, N2)
        f0 = f0_ref[:N2, :]
        f1_ref[...] = jax.nn.relu(jnp.dot(A0, wf_ref[...], precision=_HI)
                                  + jnp.dot(f0, wfs_ref[...], precision=_HI)
                                  + bf_ref[...])

    return _tc(body, jax.ShapeDtypeStruct((N2, C), jnp.float32),
               S0, c2, F0p, Wf, Wfs, bf, name="tc3_flat")


def _tc4(S1, c2, Ws1, bs1):
    def body(s_ref, c_ref, w_ref, b_ref, a1_ref, s1_ref):
        A1, _ = _combine(s_ref, c_ref, N2)
        a1_ref[...] = A1
        s1_ref[...] = jax.nn.relu(jnp.dot(A1, w_ref[...], precision=_HI)
                                  + b_ref[...])

    return _tc(body, [jax.ShapeDtypeStruct((N2, C), jnp.float32),
                      jax.ShapeDtypeStruct((N2, C), jnp.float32)],
               S1, c2, Ws1, bs1, name="tc4_skip1")


def _tc5(S2, c2, F1, Ws2, bs2):
    def body(s_ref, c_ref, f1_ref, w_ref, b_ref, skip_ref):
        A2, _ = _combine(s_ref, c_ref, N2)
        s2 = jnp.dot(A2, w_ref[...], precision=_HI) + b_ref[...]
        skip_ref[...] = jax.nn.relu(s2 + f1_ref[...])

    return _tc(body, jax.ShapeDtypeStruct((N2, C), jnp.float32),
               S2, c2, F1, Ws2, bs2, name="tc5_skip2")


def _tc6(S3, c2, A1, F1, skip, Wm1, Wm2, bm, P1, P2, Wu):
    def body(s_ref, c_ref, a1_ref, f1_ref, sk_ref, wm1_ref, wm2_ref, bm_ref,
             p1_ref, p2_ref, wu_ref, g_ref):
        A3, _ = _combine(s_ref, c_ref, N2)
        merged = jax.nn.relu(jnp.dot(a1_ref[...], wm1_ref[...], precision=_HI)
                             + jnp.dot(A3, wm2_ref[...], precision=_HI)
                             + bm_ref[...])
        fr2 = (merged
               + jnp.dot(f1_ref[...], p1_ref[...], precision=_HI)
               + jnp.dot(sk_ref[...], p2_ref[...], precision=_HI))
        g_ref[...] = jnp.dot(fr2, wu_ref[...], precision=_HI)

    return _tc(body, jax.ShapeDtypeStruct((N2, C), jnp.float32),
               S3, c2, A1, F1, skip, Wm1, Wm2, bm, P1, P2, Wu,
               name="tc6_merge")


def _tc7(G, cu, bu):
    def body(g_ref, c_ref, b_ref, out_ref):
        avg, d = _combine(g_ref, c_ref, N)
        gate = jnp.where(d > 0.0, 1.0, 0.0)
        out_ref[...] = jax.nn.relu(avg + gate * b_ref[...])

    return _tc(body, jax.ShapeDtypeStruct((N, C), jnp.float32), G, cu, bu,
               name="tc7_up")


def kernel(point_bxyz, point_feat, edge_index, sample_idx, edge_index_down,
           up_src, up_dst, W_down, W_pos, b_down, W_flat, W_flat_self, b_flat,
           W_skip1, b_skip1, W_skip2, b_skip2, W_merge, b_merge, W_up, b_up):
    f32 = jnp.float32
    src0 = jnp.pad(edge_index[0], (0, E0P - E0))
    dst0 = jnp.pad(edge_index[1], (0, E0P - E0), constant_values=N)
    src2 = jnp.pad(edge_index_down[0], (0, E1P - E1))
    dst2 = jnp.pad(edge_index_down[1], (0, E1P - E1), constant_values=N2)
    usrc = jnp.pad(up_src, (0, EUP - EU))
    udst = jnp.pad(up_dst, (0, EUP - EU), constant_values=N)
    sidx = jnp.pad(sample_idx, (0, GPAD - N2))

    bd = b_down.reshape(1, C)
    bf = b_flat.reshape(1, C)
    bs1 = b_skip1.reshape(1, C)
    bs2 = b_skip2.reshape(1, C)
    bm = b_merge.reshape(1, C)
    bu = b_up.reshape(1, C)
    Wm1, Wm2 = W_merge[:C], W_merge[C:]
    P1 = jnp.asarray(_P1)
    P2 = jnp.asarray(_P2)

    a, q = _tc1(point_bxyz, point_feat, W_down, W_pos, bd)
    qp = jnp.pad(q, ((0, NACC_N - N), (0, 0)))
    hs = _seg_call(a, qp, src0, dst0, NACC_N, name="sc_down")
    hc = _hist_call(dst0, NACC_N, name="sc_down_deg")
    h = _tc2(hs, hc)
    F0p = _gather_rows(h, sidx, name="sc_sample")
    S0 = _seg_call(F0p, None, src2, dst2, NACC_N2, name="sc_seg0")
    c2 = _hist_call(dst2, NACC_N2, name="sc_seg_deg")
    F1 = _tc3(S0, c2, F0p, W_flat, W_flat_self, bf)
    S1 = _seg_call(F1, None, src2, dst2, NACC_N2, name="sc_seg1")
    A1, s1 = _tc4(S1, c2, W_skip1, bs1)
    S2 = _seg_call(s1, None, src2, dst2, NACC_N2, name="sc_seg2")
    skip = _tc5(S2, c2, F1, W_skip2, bs2)
    S3 = _seg_call(skip, None, src2, dst2, NACC_N2, name="sc_seg3")
    g = _tc6(S3, c2, A1, F1, skip, Wm1, Wm2, bm, P1, P2, W_up)
    G = _seg_call(g, None, usrc, udst, NACC_N, name="sc_up")
    cu = _hist_call(udst, NACC_N, name="sc_up_deg")
    return _tc7(G, cu, bu)


# speedup vs baseline: 1.0826x; 1.0470x over previous
import functools

import jax
import jax.numpy as jnp
import numpy as np
from jax import lax
from jax.experimental import pallas as pl
from jax.experimental.pallas import tpu as pltpu
from jax.experimental.pallas import tpu_sc as plsc

N = 10000
N2 = 5000
E0 = 320000
E1 = 160000
EU = 30000
C = 128

NC, NS, NW = 2, 16, 32
K = 128

NACC_N = 10112
NACC_N2 = 5120
E0P = ((E0 + NW * K - 1) // (NW * K)) * (NW * K)
E1P = ((E1 + NW * K - 1) // (NW * K)) * (NW * K)
EUP = ((EU + NW * K - 1) // (NW * K)) * (NW * K)
GPAD = 5120

_HI = jax.lax.Precision.HIGHEST

_P1 = np.zeros((C, C), np.float32)
_P2 = np.zeros((C, C), np.float32)
for _j in range(C):
    _P1[_j, _j // 2] = 1.0
    _P2[_j, C // 2 + _j // 2] = 1.0

_MESH = plsc.VectorSubcoreMesh(core_axis_name="c", subcore_axis_name="s",
                               num_cores=NC, num_subcores=NS)


def _seg_call(ta, tq, src, dst, n_acc, *, name):
    two = tq is not None
    e_pad = src.shape[0]
    e_per_w = e_pad // NW
    n_chunks = e_per_w // K
    rows_sub = n_acc // NS

    out_type = jax.ShapeDtypeStruct((NC, n_acc, C), jnp.float32)

    scratch = [pltpu.VMEM((K,), jnp.int32),
               pltpu.VMEM((K,), jnp.int32),
               pltpu.VMEM((K, C), jnp.float32)]
    if two:
        scratch.append(pltpu.VMEM((K, C), jnp.float32))
    scratch.append(pltpu.VMEM_SHARED((n_acc, C), jnp.float32))
    scratch.append(pltpu.SemaphoreType.DMA)
    if two:
        scratch.append(pltpu.SemaphoreType.DMA)

    def body(*refs):
        it = iter(refs)
        ta_ref = next(it)
        tq_ref = next(it) if two else None
        src_ref = next(it)
        dst_ref = next(it)
        out_sum = next(it)
        src_v = next(it)
        dst_v = next(it)
        arows = next(it)
        qrows = next(it) if two else None
        acc = next(it)
        sem_a = next(it)
        sem_q = next(it) if two else None

        cid = lax.axis_index("c")
        sid = lax.axis_index("s")
        wid = sid * NC + cid
        r0 = sid * rows_sub

        @pl.loop(0, K)
        def _(r):
            for j in range(C // 16):
                arows[r, pl.ds(j * 16, 16)] = jnp.full((16,), 0.0,
                                                       jnp.float32)

        nfull, rem = rows_sub // K, rows_sub % K
        for k in range(nfull):
            pltpu.sync_copy(arows, acc.at[pl.ds(r0 + k * K, K)])
        if rem:
            pltpu.sync_copy(arows.at[pl.ds(0, rem)],
                            acc.at[pl.ds(r0 + nfull * K, rem)])
        plsc.subcore_barrier()

        base = wid * e_per_w

        @pl.loop(0, n_chunks)
        def _(ci):
            off = base + ci * K
            pltpu.sync_copy(src_ref.at[pl.ds(off, K)], src_v)
            pltpu.sync_copy(dst_ref.at[pl.ds(off, K)], dst_v)
            cp_a = pltpu.async_copy(ta_ref.at[src_v], arows, sem_a)
            if two:
                cp_q = pltpu.async_copy(tq_ref.at[dst_v], qrows, sem_q)
            cp_a.wait()
            if two:
                cp_q.wait()

                @pl.loop(0, K)
                def _(r):
                    for j in range(C // 16):
                        sl = pl.ds(j * 16, 16)
                        arows[r, sl] = jnp.maximum(arows[r, sl] + qrows[r, sl],
                                                   0.0)

            pltpu.sync_copy(arows, acc.at[dst_v], add=True)

        plsc.subcore_barrier()
        pltpu.sync_copy(acc.at[pl.ds(r0, rows_sub)],
                        out_sum.at[cid, pl.ds(r0, rows_sub)])

    ins = [ta] + ([tq] if two else []) + [src, dst]
    fn = pl.kernel(body, out_type=out_type, mesh=_MESH, scratch_types=scratch,
                   name=name)
    return fn(*ins)


def _hist_call(dst, n_acc, *, name):
    e_pad = dst.shape[0]
    e_per_w = e_pad // NW
    n_chunks = e_per_w // K
    rows_sub = n_acc // NS

    def body(dst_ref, out_cnt, dst_v, ones_v, cnt_acc):
        cid = lax.axis_index("c")
        sid = lax.axis_index("s")
        wid = sid * NC + cid
        r0 = sid * rows_sub

        @pl.loop(0, K)
        def _(r):
            for j in range(C // 16):
                ones_v[r, pl.ds(j * 16, 16)] = jnp.full((16,), 0.0,
                                                        jnp.float32)

        nfull, rem = rows_sub // K, rows_sub % K
        for k in range(nfull):
            pltpu.sync_copy(ones_v, cnt_acc.at[pl.ds(r0 + k * K, K)])
        if rem:
            pltpu.sync_copy(ones_v.at[pl.ds(0, rem)],
                            cnt_acc.at[pl.ds(r0 + nfull * K, rem)])

        @pl.loop(0, K)
        def _(r):
            for j in range(C // 16):
                ones_v[r, pl.ds(j * 16, 16)] = jnp.full((16,), 1.0,
                                                        jnp.float32)

        plsc.subcore_barrier()
        base = wid * e_per_w

        @pl.loop(0, n_chunks)
        def _(ci):
            off = base + ci * K
            pltpu.sync_copy(dst_ref.at[pl.ds(off, K)], dst_v)
            pltpu.sync_copy(ones_v, cnt_acc.at[dst_v], add=True)

        plsc.subcore_barrier()
        pltpu.sync_copy(cnt_acc.at[pl.ds(r0, rows_sub)],
                        out_cnt.at[cid, pl.ds(r0, rows_sub)])

    fn = pl.kernel(body,
                   out_type=jax.ShapeDtypeStruct((NC, n_acc, C), jnp.float32),
                   mesh=_MESH,
                   scratch_types=[pltpu.VMEM((K,), jnp.int32),
                                  pltpu.VMEM((K, C), jnp.float32),
                                  pltpu.VMEM_SHARED((n_acc, C), jnp.float32)],
                   name=name)
    return fn(dst)


def _sample_combine(hs0, hs1, hc0, hc1, idx, *, name):
    g = idx.shape[0]
    per_w = g // NW
    ck = next(c for c in range(min(per_w, K), 0, -8) if per_w % c == 0)
    n_chunks = per_w // ck

    def body(hs0_ref, hs1_ref, hc0_ref, hc1_ref, idx_ref, out_ref,
             idx_v, p0, p1, c0, c1, s0, s1, s2, s3):
        cid = lax.axis_index("c")
        sid = lax.axis_index("s")
        wid = sid * NC + cid
        base = wid * per_w

        @pl.loop(0, n_chunks)
        def _(ci):
            off = base + ci * ck
            pltpu.sync_copy(idx_ref.at[pl.ds(off, ck)], idx_v)
            cp0 = pltpu.async_copy(hs0_ref.at[idx_v], p0, s0)
            cp1 = pltpu.async_copy(hs1_ref.at[idx_v], p1, s1)
            cp2 = pltpu.async_copy(hc0_ref.at[idx_v], c0, s2)
            cp3 = pltpu.async_copy(hc1_ref.at[idx_v], c1, s3)
            cp0.wait()
            cp1.wait()
            cp2.wait()
            cp3.wait()

            @pl.loop(0, ck)
            def _(r):
                for j in range(C // 16):
                    sl = pl.ds(j * 16, 16)
                    d = jnp.maximum(c0[r, sl] + c1[r, sl], 1.0)
                    p0[r, sl] = (p0[r, sl] + p1[r, sl]) / d

            pltpu.sync_copy(p0, out_ref.at[pl.ds(off, ck)])

    fn = pl.kernel(body,
                   out_type=jax.ShapeDtypeStruct((g, C), jnp.float32),
                   mesh=_MESH,
                   scratch_types=[pltpu.VMEM((ck,), jnp.int32),
                                  pltpu.VMEM((ck, C), jnp.float32),
                                  pltpu.VMEM((ck, C), jnp.float32),
                                  pltpu.VMEM((ck, C), jnp.float32),
                                  pltpu.VMEM((ck, C), jnp.float32),
                                  pltpu.SemaphoreType.DMA,
                                  pltpu.SemaphoreType.DMA,
                                  pltpu.SemaphoreType.DMA,
                                  pltpu.SemaphoreType.DMA],
                   name=name)
    return fn(hs0, hs1, hc0, hc1, idx)


def _tc(fn, out_shape, *args, name):
    return pl.pallas_call(fn, out_shape=out_shape, name=name)(*args)


def _tc1(bxyz, feat, Wd, Wp, bd):
    def body(bxyz_ref, feat_ref, wd_ref, wp_ref, bd_ref, a_ref, q_ref):
        q = (bxyz_ref[:, 1:2] * wp_ref[0:1, :]
             + bxyz_ref[:, 2:3] * wp_ref[1:2, :]
             + bxyz_ref[:, 3:4] * wp_ref[2:3, :])
        a = jnp.dot(feat_ref[...], wd_ref[...], precision=_HI) + bd_ref[...] - q
        a_ref[...] = a
        q_ref[...] = q

    return _tc(body, [jax.ShapeDtypeStruct((N, C), jnp.float32),
                      jax.ShapeDtypeStruct((N, C), jnp.float32)],
               bxyz, feat, Wd, Wp, bd, name="tc1_aq")


def _combine(sums_ref, cnt_ref, n):
    s = sums_ref[0, :n, :] + sums_ref[1, :n, :]
    d = cnt_ref[0, :n, 0:1] + cnt_ref[1, :n, 0:1]
    return s / jnp.maximum(d, 1.0), d


def _tc3(S0, c2, F0p, Wf, Wfs, bf):
    def body(s_ref, c_ref, f0_ref, wf_ref, wfs_ref, bf_ref, f1_ref):
        A0, _ = _combine(s_ref, c_ref, N2)
        f0 = f0_ref[:N2, :]
        f1_ref[...] = jax.nn.relu(jnp.dot(A0, wf_ref[...], precision=_HI)
                                  + jnp.dot(f0, wfs_ref[...], precision=_HI)
                                  + bf_ref[...])

    return _tc(body, jax.ShapeDtypeStruct((N2, C), jnp.float32),
               S0, c2, F0p, Wf, Wfs, bf, name="tc3_flat")


def _tc4(S1, c2, Ws1, bs1):
    def body(s_ref, c_ref, w_ref, b_ref, a1_ref, s1_ref):
        A1, _ = _combine(s_ref, c_ref, N2)
        a1_ref[...] = A1
        s1_ref[...] = jax.nn.relu(jnp.dot(A1, w_ref[...], precision=_HI)
                                  + b_ref[...])

    return _tc(body, [jax.ShapeDtypeStruct((N2, C), jnp.float32),
                      jax.ShapeDtypeStruct((N2, C), jnp.float32)],
               S1, c2, Ws1, bs1, name="tc4_skip1")


def _tc5(S2, c2, F1, Ws2, bs2):
    def body(s_ref, c_ref, f1_ref, w_ref, b_ref, skip_ref):
        A2, _ = _combine(s_ref, c_ref, N2)
        s2 = jnp.dot(A2, w_ref[...], precision=_HI) + b_ref[...]
        skip_ref[...] = jax.nn.relu(s2 + f1_ref[...])

    return _tc(body, jax.ShapeDtypeStruct((N2, C), jnp.float32),
               S2, c2, F1, Ws2, bs2, name="tc5_skip2")


def _tc6(S3, c2, A1, F1, skip, Wm1, Wm2, bm, P1, P2, Wu):
    def body(s_ref, c_ref, a1_ref, f1_ref, sk_ref, wm1_ref, wm2_ref, bm_ref,
             p1_ref, p2_ref, wu_ref, g_ref):
        A3, _ = _combine(s_ref, c_ref, N2)
        merged = jax.nn.relu(jnp.dot(a1_ref[...], wm1_ref[...], precision=_HI)
                             + jnp.dot(A3, wm2_ref[...], precision=_HI)
                             + bm_ref[...])
        fr2 = (merged
               + jnp.dot(f1_ref[...], p1_ref[...], precision=_HI)
               + jnp.dot(sk_ref[...], p2_ref[...], precision=_HI))
        g_ref[...] = jnp.dot(fr2, wu_ref[...], precision=_HI)

    return _tc(body, jax.ShapeDtypeStruct((N2, C), jnp.float32),
               S3, c2, A1, F1, skip, Wm1, Wm2, bm, P1, P2, Wu,
               name="tc6_merge")


def _tc7(G, cu, bu):
    def body(g_ref, c_ref, b_ref, out_ref):
        avg, d = _combine(g_ref, c_ref, N)
        gate = jnp.where(d > 0.0, 1.0, 0.0)
        out_ref[...] = jax.nn.relu(avg + gate * b_ref[...])

    return _tc(body, jax.ShapeDtypeStruct((N, C), jnp.float32), G, cu, bu,
               name="tc7_up")


def kernel(point_bxyz, point_feat, edge_index, sample_idx, edge_index_down,
           up_src, up_dst, W_down, W_pos, b_down, W_flat, W_flat_self, b_flat,
           W_skip1, b_skip1, W_skip2, b_skip2, W_merge, b_merge, W_up, b_up):
    f32 = jnp.float32
    src0 = jnp.pad(edge_index[0], (0, E0P - E0))
    dst0 = jnp.pad(edge_index[1], (0, E0P - E0), constant_values=N)
    src2 = jnp.pad(edge_index_down[0], (0, E1P - E1))
    dst2 = jnp.pad(edge_index_down[1], (0, E1P - E1), constant_values=N2)
    usrc = jnp.pad(up_src, (0, EUP - EU))
    udst = jnp.pad(up_dst, (0, EUP - EU), constant_values=N)
    sidx = jnp.pad(sample_idx, (0, GPAD - N2))

    bd = b_down.reshape(1, C)
    bf = b_flat.reshape(1, C)
    bs1 = b_skip1.reshape(1, C)
    bs2 = b_skip2.reshape(1, C)
    bm = b_merge.reshape(1, C)
    bu = b_up.reshape(1, C)
    Wm1, Wm2 = W_merge[:C], W_merge[C:]
    P1 = jnp.asarray(_P1)
    P2 = jnp.asarray(_P2)

    a, q = _tc1(point_bxyz, point_feat, W_down, W_pos, bd)
    qp = jnp.pad(q, ((0, NACC_N - N), (0, 0)))
    hs = _seg_call(a, qp, src0, dst0, NACC_N, name="sc_down")
    hc = _hist_call(dst0, NACC_N, name="sc_down_deg")
    F0p = _sample_combine(hs[0], hs[1], hc[0], hc[1], sidx, name="sc_sample")
    S0 = _seg_call(F0p, None, src2, dst2, NACC_N2, name="sc_seg0")
    c2 = _hist_call(dst2, NACC_N2, name="sc_seg_deg")
    F1 = _tc3(S0, c2, F0p, W_flat, W_flat_self, bf)
    S1 = _seg_call(F1, None, src2, dst2, NACC_N2, name="sc_seg1")
    A1, s1 = _tc4(S1, c2, W_skip1, bs1)
    S2 = _seg_call(s1, None, src2, dst2, NACC_N2, name="sc_seg2")
    skip = _tc5(S2, c2, F1, W_skip2, bs2)
    S3 = _seg_call(skip, None, src2, dst2, NACC_N2, name="sc_seg3")
    g = _tc6(S3, c2, A1, F1, skip, Wm1, Wm2, bm, P1, P2, W_up)
    G = _seg_call(g, None, usrc, udst, NACC_N, name="sc_up")
    cu = _hist_call(udst, NACC_N, name="sc_up_deg")
    return _tc7(G, cu, bu)


# speedup vs baseline: 1.1488x; 1.0611x over previous
import functools

import jax
import jax.numpy as jnp
import numpy as np
from jax import lax
from jax.experimental import pallas as pl
from jax.experimental.pallas import tpu as pltpu
from jax.experimental.pallas import tpu_sc as plsc

N = 10000
N2 = 5000
E0 = 320000
E1 = 160000
EU = 30000
C = 128

NC, NS, NW = 2, 16, 32
K = 128

NACC_N = 10112
NACC_N2 = 5120
E0P = ((E0 + NW * K - 1) // (NW * K)) * (NW * K)
E1P = ((E1 + NW * K - 1) // (NW * K)) * (NW * K)
EUP = ((EU + NW * K - 1) // (NW * K)) * (NW * K)
GPAD = 5120

_HI = jax.lax.Precision.HIGHEST

_P1 = np.zeros((C, C), np.float32)
_P2 = np.zeros((C, C), np.float32)
for _j in range(C):
    _P1[_j, _j // 2] = 1.0
    _P2[_j, C // 2 + _j // 2] = 1.0

_MESH = plsc.VectorSubcoreMesh(core_axis_name="c", subcore_axis_name="s",
                               num_cores=NC, num_subcores=NS)


def _seg_call(ta, tq, src, dst, n_acc, *, name):
    two = tq is not None
    e_pad = src.shape[0]
    e_per_w = e_pad // NW
    n_chunks = e_per_w // K
    rows_sub = n_acc // NS
    pipe = (not two) and n_acc == NACC_N2

    out_type = jax.ShapeDtypeStruct((NC, n_acc, C), jnp.float32)

    scratch = [pltpu.VMEM((K,), jnp.int32),
               pltpu.VMEM((K,), jnp.int32),
               pltpu.VMEM((K, C), jnp.float32)]
    if pipe:
        scratch.append(pltpu.VMEM((K,), jnp.int32))
        scratch.append(pltpu.VMEM((K,), jnp.int32))
        scratch.append(pltpu.VMEM((K, C), jnp.float32))
    if two:
        scratch.append(pltpu.VMEM((K, C), jnp.float32))
    scratch.append(pltpu.VMEM_SHARED((n_acc, C), jnp.float32))
    scratch.append(pltpu.SemaphoreType.DMA)
    if pipe:
        scratch.append(pltpu.SemaphoreType.DMA)
    if two:
        scratch.append(pltpu.SemaphoreType.DMA)

    def body(*refs):
        it = iter(refs)
        ta_ref = next(it)
        tq_ref = next(it) if two else None
        src_ref = next(it)
        dst_ref = next(it)
        out_sum = next(it)
        src_a = next(it)
        dst_a = next(it)
        rows_a = next(it)
        if pipe:
            src_b = next(it)
            dst_b = next(it)
            rows_b = next(it)
        qrows_a = next(it) if two else None
        acc = next(it)
        sem_a0 = next(it)
        sem_a1 = next(it) if pipe else None
        sem_q0 = next(it) if two else None

        cid = lax.axis_index("c")
        sid = lax.axis_index("s")
        wid = sid * NC + cid
        r0 = sid * rows_sub

        @pl.loop(0, K)
        def _(r):
            for j in range(C // 16):
                rows_a[r, pl.ds(j * 16, 16)] = jnp.full((16,), 0.0,
                                                        jnp.float32)

        nfull, rem = rows_sub // K, rows_sub % K
        for k in range(nfull):
            pltpu.sync_copy(rows_a, acc.at[pl.ds(r0 + k * K, K)])
        if rem:
            pltpu.sync_copy(rows_a.at[pl.ds(0, rem)],
                            acc.at[pl.ds(r0 + nfull * K, rem)])
        plsc.subcore_barrier()

        base = wid * e_per_w

        def issue(off, src_v, dst_v, rows_v, qrows_v, sa, sq):
            pltpu.sync_copy(src_ref.at[pl.ds(off, K)], src_v)
            pltpu.sync_copy(dst_ref.at[pl.ds(off, K)], dst_v)
            cps = [pltpu.async_copy(ta_ref.at[src_v], rows_v, sa)]
            if two:
                cps.append(pltpu.async_copy(tq_ref.at[dst_v], qrows_v, sq))
            return cps

        def finish(cps, dst_v, rows_v, qrows_v):
            for cp in cps:
                cp.wait()
            if two:
                @pl.loop(0, K)
                def _(r):
                    for j in range(C // 16):
                        sl = pl.ds(j * 16, 16)
                        rows_v[r, sl] = jnp.maximum(
                            rows_v[r, sl] + qrows_v[r, sl], 0.0)

            pltpu.sync_copy(rows_v, acc.at[dst_v], add=True)

        if pipe:
            @pl.loop(0, n_chunks // 2)
            def _(h):
                off0 = base + (2 * h) * K
                cps_a = issue(off0, src_a, dst_a, rows_a, None,
                              sem_a0, None)
                cps_b = issue(off0 + K, src_b, dst_b, rows_b, None,
                              sem_a1, None)
                finish(cps_a, dst_a, rows_a, None)
                finish(cps_b, dst_b, rows_b, None)

            if n_chunks % 2:
                off_l = base + (n_chunks - 1) * K
                cps_l = issue(off_l, src_a, dst_a, rows_a, None,
                              sem_a0, None)
                finish(cps_l, dst_a, rows_a, None)
        else:
            @pl.loop(0, n_chunks)
            def _(ci):
                off = base + ci * K
                cps = issue(off, src_a, dst_a, rows_a, qrows_a,
                            sem_a0, sem_q0)
                finish(cps, dst_a, rows_a, qrows_a)

        plsc.subcore_barrier()
        pltpu.sync_copy(acc.at[pl.ds(r0, rows_sub)],
                        out_sum.at[cid, pl.ds(r0, rows_sub)])

    ins = [ta] + ([tq] if two else []) + [src, dst]
    fn = pl.kernel(body, out_type=out_type, mesh=_MESH, scratch_types=scratch,
                   name=name)
    return fn(*ins)


def _hist_call(dst, n_acc, *, name):
    e_pad = dst.shape[0]
    e_per_w = e_pad // NW
    n_chunks = e_per_w // K
    rows_sub = n_acc // NS

    def body(dst_ref, out_cnt, dst_v, ones_v, cnt_acc):
        cid = lax.axis_index("c")
        sid = lax.axis_index("s")
        wid = sid * NC + cid
        r0 = sid * rows_sub

        @pl.loop(0, K)
        def _(r):
            for j in range(C // 16):
                ones_v[r, pl.ds(j * 16, 16)] = jnp.full((16,), 0.0,
                                                        jnp.float32)

        nfull, rem = rows_sub // K, rows_sub % K
        for k in range(nfull):
            pltpu.sync_copy(ones_v, cnt_acc.at[pl.ds(r0 + k * K, K)])
        if rem:
            pltpu.sync_copy(ones_v.at[pl.ds(0, rem)],
                            cnt_acc.at[pl.ds(r0 + nfull * K, rem)])

        @pl.loop(0, K)
        def _(r):
            for j in range(C // 16):
                ones_v[r, pl.ds(j * 16, 16)] = jnp.full((16,), 1.0,
                                                        jnp.float32)

        plsc.subcore_barrier()
        base = wid * e_per_w

        @pl.loop(0, n_chunks)
        def _(ci):
            off = base + ci * K
            pltpu.sync_copy(dst_ref.at[pl.ds(off, K)], dst_v)
            pltpu.sync_copy(ones_v, cnt_acc.at[dst_v], add=True)

        plsc.subcore_barrier()
        pltpu.sync_copy(cnt_acc.at[pl.ds(r0, rows_sub)],
                        out_cnt.at[cid, pl.ds(r0, rows_sub)])

    fn = pl.kernel(body,
                   out_type=jax.ShapeDtypeStruct((NC, n_acc, C), jnp.float32),
                   mesh=_MESH,
                   scratch_types=[pltpu.VMEM((K,), jnp.int32),
                                  pltpu.VMEM((K, C), jnp.float32),
                                  pltpu.VMEM_SHARED((n_acc, C), jnp.float32)],
                   name=name)
    return fn(dst)


def _sample_combine(hs0, hs1, hc0, hc1, idx, *, name):
    g = idx.shape[0]
    per_w = g // NW
    ck = next(c for c in range(min(per_w, K), 0, -8) if per_w % c == 0)
    n_chunks = per_w // ck

    def body(hs0_ref, hs1_ref, hc0_ref, hc1_ref, idx_ref, out_ref,
             idx_v, p0, p1, c0, c1, s0, s1, s2, s3):
        cid = lax.axis_index("c")
        sid = lax.axis_index("s")
        wid = sid * NC + cid
        base = wid * per_w

        @pl.loop(0, n_chunks)
        def _(ci):
            off = base + ci * ck
            pltpu.sync_copy(idx_ref.at[pl.ds(off, ck)], idx_v)
            cp0 = pltpu.async_copy(hs0_ref.at[idx_v], p0, s0)
            cp1 = pltpu.async_copy(hs1_ref.at[idx_v], p1, s1)
            cp2 = pltpu.async_copy(hc0_ref.at[idx_v], c0, s2)
            cp3 = pltpu.async_copy(hc1_ref.at[idx_v], c1, s3)
            cp0.wait()
            cp1.wait()
            cp2.wait()
            cp3.wait()

            @pl.loop(0, ck)
            def _(r):
                for j in range(C // 16):
                    sl = pl.ds(j * 16, 16)
                    d = jnp.maximum(c0[r, sl] + c1[r, sl], 1.0)
                    p0[r, sl] = (p0[r, sl] + p1[r, sl]) / d

            pltpu.sync_copy(p0, out_ref.at[pl.ds(off, ck)])

    fn = pl.kernel(body,
                   out_type=jax.ShapeDtypeStruct((g, C), jnp.float32),
                   mesh=_MESH,
                   scratch_types=[pltpu.VMEM((ck,), jnp.int32),
                                  pltpu.VMEM((ck, C), jnp.float32),
                                  pltpu.VMEM((ck, C), jnp.float32),
                                  pltpu.VMEM((ck, C), jnp.float32),
                                  pltpu.VMEM((ck, C), jnp.float32),
                                  pltpu.SemaphoreType.DMA,
                                  pltpu.SemaphoreType.DMA,
                                  pltpu.SemaphoreType.DMA,
                                  pltpu.SemaphoreType.DMA],
                   name=name)
    return fn(hs0, hs1, hc0, hc1, idx)


def _tc(fn, out_shape, *args, name):
    return pl.pallas_call(fn, out_shape=out_shape, name=name)(*args)


def _tc1(bxyz, feat, Wd, Wp, bd):
    def body(bxyz_ref, feat_ref, wd_ref, wp_ref, bd_ref, a_ref, q_ref):
        q = (bxyz_ref[:, 1:2] * wp_ref[0:1, :]
             + bxyz_ref[:, 2:3] * wp_ref[1:2, :]
             + bxyz_ref[:, 3:4] * wp_ref[2:3, :])
        a = jnp.dot(feat_ref[...], wd_ref[...], precision=_HI) + bd_ref[...] - q
        a_ref[...] = a
        q_ref[...] = q

    return _tc(body, [jax.ShapeDtypeStruct((N, C), jnp.float32),
                      jax.ShapeDtypeStruct((N, C), jnp.float32)],
               bxyz, feat, Wd, Wp, bd, name="tc1_aq")


def _combine(sums_ref, cnt_ref, n):
    s = sums_ref[0, :n, :] + sums_ref[1, :n, :]
    d = cnt_ref[0, :n, 0:1] + cnt_ref[1, :n, 0:1]
    return s / jnp.maximum(d, 1.0), d


def _tc3(S0, c2, F0p, Wf, Wfs, bf):
    def body(s_ref, c_ref, f0_ref, wf_ref, wfs_ref, bf_ref, f1_ref):
        A0, _ = _combine(s_ref, c_ref, N2)
        f0 = f0_ref[:N2, :]
        f1_ref[...] = jax.nn.relu(jnp.dot(A0, wf_ref[...], precision=_HI)
                                  + jnp.dot(f0, wfs_ref[...], precision=_HI)
                                  + bf_ref[...])

    return _tc(body, jax.ShapeDtypeStruct((N2, C), jnp.float32),
               S0, c2, F0p, Wf, Wfs, bf, name="tc3_flat")


def _tc4(S1, c2, Ws1, bs1):
    def body(s_ref, c_ref, w_ref, b_ref, a1_ref, s1_ref):
        A1, _ = _combine(s_ref, c_ref, N2)
        a1_ref[...] = A1
        s1_ref[...] = jax.nn.relu(jnp.dot(A1, w_ref[...], precision=_HI)
                                  + b_ref[...])

    return _tc(body, [jax.ShapeDtypeStruct((N2, C), jnp.float32),
                      jax.ShapeDtypeStruct((N2, C), jnp.float32)],
               S1, c2, Ws1, bs1, name="tc4_skip1")


def _tc5(S2, c2, F1, Ws2, bs2):
    def body(s_ref, c_ref, f1_ref, w_ref, b_ref, skip_ref):
        A2, _ = _combine(s_ref, c_ref, N2)
        s2 = jnp.dot(A2, w_ref[...], precision=_HI) + b_ref[...]
        skip_ref[...] = jax.nn.relu(s2 + f1_ref[...])

    return _tc(body, jax.ShapeDtypeStruct((N2, C), jnp.float32),
               S2, c2, F1, Ws2, bs2, name="tc5_skip2")


def _tc6(S3, c2, A1, F1, skip, Wm1, Wm2, bm, P1, P2, Wu):
    def body(s_ref, c_ref, a1_ref, f1_ref, sk_ref, wm1_ref, wm2_ref, bm_ref,
             p1_ref, p2_ref, wu_ref, g_ref):
        A3, _ = _combine(s_ref, c_ref, N2)
        merged = jax.nn.relu(jnp.dot(a1_ref[...], wm1_ref[...], precision=_HI)
                             + jnp.dot(A3, wm2_ref[...], precision=_HI)
                             + bm_ref[...])
        fr2 = (merged
               + jnp.dot(f1_ref[...], p1_ref[...], precision=_HI)
               + jnp.dot(sk_ref[...], p2_ref[...], precision=_HI))
        g_ref[...] = jnp.dot(fr2, wu_ref[...], precision=_HI)

    return _tc(body, jax.ShapeDtypeStruct((N2, C), jnp.float32),
               S3, c2, A1, F1, skip, Wm1, Wm2, bm, P1, P2, Wu,
               name="tc6_merge")


def _tc7(G, cu, bu):
    def body(g_ref, c_ref, b_ref, out_ref):
        avg, d = _combine(g_ref, c_ref, N)
        gate = jnp.where(d > 0.0, 1.0, 0.0)
        out_ref[...] = jax.nn.relu(avg + gate * b_ref[...])

    return _tc(body, jax.ShapeDtypeStruct((N, C), jnp.float32), G, cu, bu,
               name="tc7_up")


def kernel(point_bxyz, point_feat, edge_index, sample_idx, edge_index_down,
           up_src, up_dst, W_down, W_pos, b_down, W_flat, W_flat_self, b_flat,
           W_skip1, b_skip1, W_skip2, b_skip2, W_merge, b_merge, W_up, b_up):
    f32 = jnp.float32
    src0 = jnp.pad(edge_index[0], (0, E0P - E0))
    dst0 = jnp.pad(edge_index[1], (0, E0P - E0), constant_values=N)
    src2 = jnp.pad(edge_index_down[0], (0, E1P - E1))
    dst2 = jnp.pad(edge_index_down[1], (0, E1P - E1), constant_values=N2)
    usrc = jnp.pad(up_src, (0, EUP - EU))
    udst = jnp.pad(up_dst, (0, EUP - EU), constant_values=N)
    sidx = jnp.pad(sample_idx, (0, GPAD - N2))

    bd = b_down.reshape(1, C)
    bf = b_flat.reshape(1, C)
    bs1 = b_skip1.reshape(1, C)
    bs2 = b_skip2.reshape(1, C)
    bm = b_merge.reshape(1, C)
    bu = b_up.reshape(1, C)
    Wm1, Wm2 = W_merge[:C], W_merge[C:]
    P1 = jnp.asarray(_P1)
    P2 = jnp.asarray(_P2)

    a, q = _tc1(point_bxyz, point_feat, W_down, W_pos, bd)
    qp = jnp.pad(q, ((0, NACC_N - N), (0, 0)))
    hs = _seg_call(a, qp, src0, dst0, NACC_N, name="sc_down")
    hc = _hist_call(dst0, NACC_N, name="sc_down_deg")
    F0p = _sample_combine(hs[0], hs[1], hc[0], hc[1], sidx, name="sc_sample")
    S0 = _seg_call(F0p, None, src2, dst2, NACC_N2, name="sc_seg0")
    c2 = _hist_call(dst2, NACC_N2, name="sc_seg_deg")
    F1 = _tc3(S0, c2, F0p, W_flat, W_flat_self, bf)
    S1 = _seg_call(F1, None, src2, dst2, NACC_N2, name="sc_seg1")
    A1, s1 = _tc4(S1, c2, W_skip1, bs1)
    S2 = _seg_call(s1, None, src2, dst2, NACC_N2, name="sc_seg2")
    skip = _tc5(S2, c2, F1, W_skip2, bs2)
    S3 = _seg_call(skip, None, src2, dst2, NACC_N2, name="sc_seg3")
    g = _tc6(S3, c2, A1, F1, skip, Wm1, Wm2, bm, P1, P2, W_up)
    G = _seg_call(g, None, usrc, udst, NACC_N, name="sc_up")
    cu = _hist_call(udst, NACC_N, name="sc_up_deg")
    return _tc7(G, cu, bu)


# speedup vs baseline: 1.1934x; 1.0389x over previous
import functools

import jax
import jax.numpy as jnp
import numpy as np
from jax import lax
from jax.experimental import pallas as pl
from jax.experimental.pallas import tpu as pltpu
from jax.experimental.pallas import tpu_sc as plsc

N = 10000
N2 = 5000
E0 = 320000
E1 = 160000
EU = 30000
C = 128

NC, NS, NW = 2, 16, 32
K = 128

NACC_N = 10112
NACC_N2 = 5120
E0P = ((E0 + NW * K - 1) // (NW * K)) * (NW * K)
E1P = ((E1 + NW * K - 1) // (NW * K)) * (NW * K)
EUP = ((EU + NW * K - 1) // (NW * K)) * (NW * K)
GPAD = 5120

_HI = jax.lax.Precision.HIGHEST

_P1 = np.zeros((C, C), np.float32)
_P2 = np.zeros((C, C), np.float32)
for _j in range(C):
    _P1[_j, _j // 2] = 1.0
    _P2[_j, C // 2 + _j // 2] = 1.0

_MESH = plsc.VectorSubcoreMesh(core_axis_name="c", subcore_axis_name="s",
                               num_cores=NC, num_subcores=NS)


def _seg_call(ta, tq, src, dst, n_acc, *, name):
    two = tq is not None
    e_pad = src.shape[0]
    e_per_w = e_pad // NW
    rows_sub = n_acc // NS
    ck = K if (not two and n_acc == NACC_N2) else K // 2
    n_chunks = e_per_w // ck

    out_type = jax.ShapeDtypeStruct((NC, n_acc, C), jnp.float32)

    scratch = [pltpu.VMEM((ck,), jnp.int32),
               pltpu.VMEM((ck,), jnp.int32),
               pltpu.VMEM((ck, C), jnp.float32),
               pltpu.VMEM((ck,), jnp.int32),
               pltpu.VMEM((ck,), jnp.int32),
               pltpu.VMEM((ck, C), jnp.float32)]
    if two:
        scratch.append(pltpu.VMEM((ck, C), jnp.float32))
        scratch.append(pltpu.VMEM((ck, C), jnp.float32))
    scratch.append(pltpu.VMEM_SHARED((n_acc, C), jnp.float32))
    scratch.append(pltpu.SemaphoreType.DMA)
    scratch.append(pltpu.SemaphoreType.DMA)
    if two:
        scratch.append(pltpu.SemaphoreType.DMA)
        scratch.append(pltpu.SemaphoreType.DMA)

    def body(*refs):
        it = iter(refs)
        ta_ref = next(it)
        tq_ref = next(it) if two else None
        src_ref = next(it)
        dst_ref = next(it)
        out_sum = next(it)
        src_a = next(it)
        dst_a = next(it)
        rows_a = next(it)
        src_b = next(it)
        dst_b = next(it)
        rows_b = next(it)
        if two:
            qrows_a = next(it)
            qrows_b = next(it)
        else:
            qrows_a = qrows_b = None
        acc = next(it)
        sem_a0 = next(it)
        sem_a1 = next(it)
        if two:
            sem_q0 = next(it)
            sem_q1 = next(it)
        else:
            sem_q0 = sem_q1 = None

        cid = lax.axis_index("c")
        sid = lax.axis_index("s")
        wid = sid * NC + cid
        r0 = sid * rows_sub

        @pl.loop(0, ck)
        def _(r):
            for j in range(C // 16):
                rows_a[r, pl.ds(j * 16, 16)] = jnp.full((16,), 0.0,
                                                        jnp.float32)

        nfull, rem = rows_sub // ck, rows_sub % ck
        for k in range(nfull):
            pltpu.sync_copy(rows_a, acc.at[pl.ds(r0 + k * ck, ck)])
        if rem:
            pltpu.sync_copy(rows_a.at[pl.ds(0, rem)],
                            acc.at[pl.ds(r0 + nfull * ck, rem)])
        plsc.subcore_barrier()

        base = wid * e_per_w

        def issue(off, src_v, dst_v, rows_v, qrows_v, sa, sq):
            pltpu.sync_copy(src_ref.at[pl.ds(off, ck)], src_v)
            pltpu.sync_copy(dst_ref.at[pl.ds(off, ck)], dst_v)
            cps = [pltpu.async_copy(ta_ref.at[src_v], rows_v, sa)]
            if two:
                cps.append(pltpu.async_copy(tq_ref.at[dst_v], qrows_v, sq))
            return cps

        def finish(cps, dst_v, rows_v, qrows_v):
            for cp in cps:
                cp.wait()
            if two:
                @pl.loop(0, ck)
                def _(r):
                    for j in range(C // 16):
                        sl = pl.ds(j * 16, 16)
                        rows_v[r, sl] = jnp.maximum(
                            rows_v[r, sl] + qrows_v[r, sl], 0.0)

            pltpu.sync_copy(rows_v, acc.at[dst_v], add=True)

        @pl.loop(0, n_chunks // 2)
        def _(h):
            off0 = base + (2 * h) * ck
            cps_a = issue(off0, src_a, dst_a, rows_a, qrows_a,
                          sem_a0, sem_q0)
            cps_b = issue(off0 + ck, src_b, dst_b, rows_b, qrows_b,
                          sem_a1, sem_q1)
            finish(cps_a, dst_a, rows_a, qrows_a)
            finish(cps_b, dst_b, rows_b, qrows_b)

        if n_chunks % 2:
            off_l = base + (n_chunks - 1) * ck
            cps_l = issue(off_l, src_a, dst_a, rows_a, qrows_a,
                          sem_a0, sem_q0)
            finish(cps_l, dst_a, rows_a, qrows_a)

        plsc.subcore_barrier()
        pltpu.sync_copy(acc.at[pl.ds(r0, rows_sub)],
                        out_sum.at[cid, pl.ds(r0, rows_sub)])

    ins = [ta] + ([tq] if two else []) + [src, dst]
    fn = pl.kernel(body, out_type=out_type, mesh=_MESH, scratch_types=scratch,
                   name=name)
    return fn(*ins)


def _hist_call(dst, n_acc, *, name):
    e_pad = dst.shape[0]
    e_per_w = e_pad // NW
    n_chunks = e_per_w // K
    rows_sub = n_acc // NS

    def body(dst_ref, out_cnt, dst_v, ones_v, cnt_acc):
        cid = lax.axis_index("c")
        sid = lax.axis_index("s")
        wid = sid * NC + cid
        r0 = sid * rows_sub

        @pl.loop(0, K)
        def _(r):
            for j in range(C // 16):
                ones_v[r, pl.ds(j * 16, 16)] = jnp.full((16,), 0.0,
                                                        jnp.float32)

        nfull, rem = rows_sub // K, rows_sub % K
        for k in range(nfull):
            pltpu.sync_copy(ones_v, cnt_acc.at[pl.ds(r0 + k * K, K)])
        if rem:
            pltpu.sync_copy(ones_v.at[pl.ds(0, rem)],
                            cnt_acc.at[pl.ds(r0 + nfull * K, rem)])

        @pl.loop(0, K)
        def _(r):
            for j in range(C // 16):
                ones_v[r, pl.ds(j * 16, 16)] = jnp.full((16,), 1.0,
                                                        jnp.float32)

        plsc.subcore_barrier()
        base = wid * e_per_w

        @pl.loop(0, n_chunks)
        def _(ci):
            off = base + ci * K
            pltpu.sync_copy(dst_ref.at[pl.ds(off, K)], dst_v)
            pltpu.sync_copy(ones_v, cnt_acc.at[dst_v], add=True)

        plsc.subcore_barrier()
        pltpu.sync_copy(cnt_acc.at[pl.ds(r0, rows_sub)],
                        out_cnt.at[cid, pl.ds(r0, rows_sub)])

    fn = pl.kernel(body,
                   out_type=jax.ShapeDtypeStruct((NC, n_acc, C), jnp.float32),
                   mesh=_MESH,
                   scratch_types=[pltpu.VMEM((K,), jnp.int32),
                                  pltpu.VMEM((K, C), jnp.float32),
                                  pltpu.VMEM_SHARED((n_acc, C), jnp.float32)],
                   name=name)
    return fn(dst)


def _sample_combine(hs0, hs1, hc0, hc1, idx, *, name):
    g = idx.shape[0]
    per_w = g // NW
    ck = next(c for c in range(min(per_w, K), 0, -8) if per_w % c == 0)
    n_chunks = per_w // ck

    def body(hs0_ref, hs1_ref, hc0_ref, hc1_ref, idx_ref, out_ref,
             idx_v, p0, p1, c0, c1, s0, s1, s2, s3):
        cid = lax.axis_index("c")
        sid = lax.axis_index("s")
        wid = sid * NC + cid
        base = wid * per_w

        @pl.loop(0, n_chunks)
        def _(ci):
            off = base + ci * ck
            pltpu.sync_copy(idx_ref.at[pl.ds(off, ck)], idx_v)
            cp0 = pltpu.async_copy(hs0_ref.at[idx_v], p0, s0)
            cp1 = pltpu.async_copy(hs1_ref.at[idx_v], p1, s1)
            cp2 = pltpu.async_copy(hc0_ref.at[idx_v], c0, s2)
            cp3 = pltpu.async_copy(hc1_ref.at[idx_v], c1, s3)
            cp0.wait()
            cp1.wait()
            cp2.wait()
            cp3.wait()

            @pl.loop(0, ck)
            def _(r):
                for j in range(C // 16):
                    sl = pl.ds(j * 16, 16)
                    d = jnp.maximum(c0[r, sl] + c1[r, sl], 1.0)
                    p0[r, sl] = (p0[r, sl] + p1[r, sl]) / d

            pltpu.sync_copy(p0, out_ref.at[pl.ds(off, ck)])

    fn = pl.kernel(body,
                   out_type=jax.ShapeDtypeStruct((g, C), jnp.float32),
                   mesh=_MESH,
                   scratch_types=[pltpu.VMEM((ck,), jnp.int32),
                                  pltpu.VMEM((ck, C), jnp.float32),
                                  pltpu.VMEM((ck, C), jnp.float32),
                                  pltpu.VMEM((ck, C), jnp.float32),
                                  pltpu.VMEM((ck, C), jnp.float32),
                                  pltpu.SemaphoreType.DMA,
                                  pltpu.SemaphoreType.DMA,
                                  pltpu.SemaphoreType.DMA,
                                  pltpu.SemaphoreType.DMA],
                   name=name)
    return fn(hs0, hs1, hc0, hc1, idx)


def _tc(fn, out_shape, *args, name):
    return pl.pallas_call(fn, out_shape=out_shape, name=name)(*args)


def _tc1(bxyz, feat, Wd, Wp, bd):
    def body(bxyz_ref, feat_ref, wd_ref, wp_ref, bd_ref, a_ref, q_ref):
        q = (bxyz_ref[:, 1:2] * wp_ref[0:1, :]
             + bxyz_ref[:, 2:3] * wp_ref[1:2, :]
             + bxyz_ref[:, 3:4] * wp_ref[2:3, :])
        a = jnp.dot(feat_ref[...], wd_ref[...], precision=_HI) + bd_ref[...] - q
        a_ref[...] = a
        q_ref[...] = q

    return _tc(body, [jax.ShapeDtypeStruct((N, C), jnp.float32),
                      jax.ShapeDtypeStruct((N, C), jnp.float32)],
               bxyz, feat, Wd, Wp, bd, name="tc1_aq")


def _combine(sums_ref, cnt_ref, n):
    s = sums_ref[0, :n, :] + sums_ref[1, :n, :]
    d = cnt_ref[0, :n, 0:1] + cnt_ref[1, :n, 0:1]
    return s / jnp.maximum(d, 1.0), d


def _tc3(S0, c2, F0p, Wf, Wfs, bf):
    def body(s_ref, c_ref, f0_ref, wf_ref, wfs_ref, bf_ref, f1_ref):
        A0, _ = _combine(s_ref, c_ref, N2)
        f0 = f0_ref[:N2, :]
        f1_ref[...] = jax.nn.relu(jnp.dot(A0, wf_ref[...], precision=_HI)
                                  + jnp.dot(f0, wfs_ref[...], precision=_HI)
                                  + bf_ref[...])

    return _tc(body, jax.ShapeDtypeStruct((N2, C), jnp.float32),
               S0, c2, F0p, Wf, Wfs, bf, name="tc3_flat")


def _tc4(S1, c2, Ws1, bs1):
    def body(s_ref, c_ref, w_ref, b_ref, a1_ref, s1_ref):
        A1, _ = _combine(s_ref, c_ref, N2)
        a1_ref[...] = A1
        s1_ref[...] = jax.nn.relu(jnp.dot(A1, w_ref[...], precision=_HI)
                                  + b_ref[...])

    return _tc(body, [jax.ShapeDtypeStruct((N2, C), jnp.float32),
                      jax.ShapeDtypeStruct((N2, C), jnp.float32)],
               S1, c2, Ws1, bs1, name="tc4_skip1")


def _tc5(S2, c2, F1, Ws2, bs2):
    def body(s_ref, c_ref, f1_ref, w_ref, b_ref, skip_ref):
        A2, _ = _combine(s_ref, c_ref, N2)
        s2 = jnp.dot(A2, w_ref[...], precision=_HI) + b_ref[...]
        skip_ref[...] = jax.nn.relu(s2 + f1_ref[...])

    return _tc(body, jax.ShapeDtypeStruct((N2, C), jnp.float32),
               S2, c2, F1, Ws2, bs2, name="tc5_skip2")


def _tc6(S3, c2, A1, F1, skip, Wm1, Wm2, bm, P1, P2, Wu):
    def body(s_ref, c_ref, a1_ref, f1_ref, sk_ref, wm1_ref, wm2_ref, bm_ref,
             p1_ref, p2_ref, wu_ref, g_ref):
        A3, _ = _combine(s_ref, c_ref, N2)
        merged = jax.nn.relu(jnp.dot(a1_ref[...], wm1_ref[...], precision=_HI)
                             + jnp.dot(A3, wm2_ref[...], precision=_HI)
                             + bm_ref[...])
        fr2 = (merged
               + jnp.dot(f1_ref[...], p1_ref[...], precision=_HI)
               + jnp.dot(sk_ref[...], p2_ref[...], precision=_HI))
        g_ref[...] = jnp.dot(fr2, wu_ref[...], precision=_HI)

    return _tc(body, jax.ShapeDtypeStruct((N2, C), jnp.float32),
               S3, c2, A1, F1, skip, Wm1, Wm2, bm, P1, P2, Wu,
               name="tc6_merge")


def _tc7(G, cu, bu):
    def body(g_ref, c_ref, b_ref, out_ref):
        avg, d = _combine(g_ref, c_ref, N)
        gate = jnp.where(d > 0.0, 1.0, 0.0)
        out_ref[...] = jax.nn.relu(avg + gate * b_ref[...])

    return _tc(body, jax.ShapeDtypeStruct((N, C), jnp.float32), G, cu, bu,
               name="tc7_up")


def kernel(point_bxyz, point_feat, edge_index, sample_idx, edge_index_down,
           up_src, up_dst, W_down, W_pos, b_down, W_flat, W_flat_self, b_flat,
           W_skip1, b_skip1, W_skip2, b_skip2, W_merge, b_merge, W_up, b_up):
    f32 = jnp.float32
    src0 = jnp.pad(edge_index[0], (0, E0P - E0))
    dst0 = jnp.pad(edge_index[1], (0, E0P - E0), constant_values=N)
    src2 = jnp.pad(edge_index_down[0], (0, E1P - E1))
    dst2 = jnp.pad(edge_index_down[1], (0, E1P - E1), constant_values=N2)
    usrc = jnp.pad(up_src, (0, EUP - EU))
    udst = jnp.pad(up_dst, (0, EUP - EU), constant_values=N)
    sidx = jnp.pad(sample_idx, (0, GPAD - N2))

    bd = b_down.reshape(1, C)
    bf = b_flat.reshape(1, C)
    bs1 = b_skip1.reshape(1, C)
    bs2 = b_skip2.reshape(1, C)
    bm = b_merge.reshape(1, C)
    bu = b_up.reshape(1, C)
    Wm1, Wm2 = W_merge[:C], W_merge[C:]
    P1 = jnp.asarray(_P1)
    P2 = jnp.asarray(_P2)

    a, q = _tc1(point_bxyz, point_feat, W_down, W_pos, bd)
    qp = jnp.pad(q, ((0, NACC_N - N), (0, 0)))
    hs = _seg_call(a, qp, src0, dst0, NACC_N, name="sc_down")
    hc = _hist_call(dst0, NACC_N, name="sc_down_deg")
    F0p = _sample_combine(hs[0], hs[1], hc[0], hc[1], sidx, name="sc_sample")
    S0 = _seg_call(F0p, None, src2, dst2, NACC_N2, name="sc_seg0")
    c2 = _hist_call(dst2, NACC_N2, name="sc_seg_deg")
    F1 = _tc3(S0, c2, F0p, W_flat, W_flat_self, bf)
    S1 = _seg_call(F1, None, src2, dst2, NACC_N2, name="sc_seg1")
    A1, s1 = _tc4(S1, c2, W_skip1, bs1)
    S2 = _seg_call(s1, None, src2, dst2, NACC_N2, name="sc_seg2")
    skip = _tc5(S2, c2, F1, W_skip2, bs2)
    S3 = _seg_call(skip, None, src2, dst2, NACC_N2, name="sc_seg3")
    g = _tc6(S3, c2, A1, F1, skip, Wm1, Wm2, bm, P1, P2, W_up)
    G = _seg_call(g, None, usrc, udst, NACC_N, name="sc_up")
    cu = _hist_call(udst, NACC_N, name="sc_up_deg")
    return _tc7(G, cu, bu)


# speedup vs baseline: 1.2039x; 1.0088x over previous
import functools

import jax
import jax.numpy as jnp
import numpy as np
from jax import lax
from jax.experimental import pallas as pl
from jax.experimental.pallas import tpu as pltpu
from jax.experimental.pallas import tpu_sc as plsc

N = 10000
N2 = 5000
E0 = 320000
E1 = 160000
EU = 30000
C = 128

NC, NS, NW = 2, 16, 32
K = 128

NACC_N = 10112
NACC_N2 = 5120
E0P = ((E0 + NW * K - 1) // (NW * K)) * (NW * K)
E1P = ((E1 + NW * K - 1) // (NW * K)) * (NW * K)
EUP = ((EU + NW * K - 1) // (NW * K)) * (NW * K)
GPAD = 5120

_HI = jax.lax.Precision.HIGHEST

_P1 = np.zeros((C, C), np.float32)
_P2 = np.zeros((C, C), np.float32)
for _j in range(C):
    _P1[_j, _j // 2] = 1.0
    _P2[_j, C // 2 + _j // 2] = 1.0

_MESH = plsc.VectorSubcoreMesh(core_axis_name="c", subcore_axis_name="s",
                               num_cores=NC, num_subcores=NS)


def _seg_call(ta, tq, src, dst, n_acc, *, name):
    two = tq is not None
    e_pad = src.shape[0]
    e_per_w = e_pad // NW
    rows_sub = n_acc // NS
    ck = K if (not two and n_acc == NACC_N2) else K // 2
    n_chunks = e_per_w // ck

    out_type = jax.ShapeDtypeStruct((NC, n_acc, C), jnp.float32)

    scratch = [pltpu.VMEM((ck,), jnp.int32),
               pltpu.VMEM((ck,), jnp.int32),
               pltpu.VMEM((ck, C), jnp.float32),
               pltpu.VMEM((ck,), jnp.int32),
               pltpu.VMEM((ck,), jnp.int32),
               pltpu.VMEM((ck, C), jnp.float32)]
    if two:
        scratch.append(pltpu.VMEM((ck, C), jnp.float32))
        scratch.append(pltpu.VMEM((ck, C), jnp.float32))
    scratch.append(pltpu.VMEM_SHARED((n_acc, C), jnp.float32))
    scratch.append(pltpu.SemaphoreType.DMA)
    scratch.append(pltpu.SemaphoreType.DMA)
    if two:
        scratch.append(pltpu.SemaphoreType.DMA)
        scratch.append(pltpu.SemaphoreType.DMA)

    def body(*refs):
        it = iter(refs)
        ta_ref = next(it)
        tq_ref = next(it) if two else None
        src_ref = next(it)
        dst_ref = next(it)
        out_sum = next(it)
        src_a = next(it)
        dst_a = next(it)
        rows_a = next(it)
        src_b = next(it)
        dst_b = next(it)
        rows_b = next(it)
        if two:
            qrows_a = next(it)
            qrows_b = next(it)
        else:
            qrows_a = qrows_b = None
        acc = next(it)
        sem_a0 = next(it)
        sem_a1 = next(it)
        if two:
            sem_q0 = next(it)
            sem_q1 = next(it)
        else:
            sem_q0 = sem_q1 = None

        cid = lax.axis_index("c")
        sid = lax.axis_index("s")
        wid = sid * NC + cid
        r0 = sid * rows_sub

        @pl.loop(0, ck)
        def _(r):
            for j in range(C // 16):
                rows_a[r, pl.ds(j * 16, 16)] = jnp.full((16,), 0.0,
                                                        jnp.float32)

        nfull, rem = rows_sub // ck, rows_sub % ck
        for k in range(nfull):
            pltpu.sync_copy(rows_a, acc.at[pl.ds(r0 + k * ck, ck)])
        if rem:
            pltpu.sync_copy(rows_a.at[pl.ds(0, rem)],
                            acc.at[pl.ds(r0 + nfull * ck, rem)])
        plsc.subcore_barrier()

        base = wid * e_per_w

        def issue(off, src_v, dst_v, rows_v, qrows_v, sa, sq):
            pltpu.sync_copy(src_ref.at[pl.ds(off, ck)], src_v)
            pltpu.sync_copy(dst_ref.at[pl.ds(off, ck)], dst_v)
            cps = [pltpu.async_copy(ta_ref.at[src_v], rows_v, sa)]
            if two:
                cps.append(pltpu.async_copy(tq_ref.at[dst_v], qrows_v, sq))
            return cps

        def finish(cps, dst_v, rows_v, qrows_v):
            for cp in cps:
                cp.wait()
            if two:
                @pl.loop(0, ck)
                def _(r):
                    for j in range(C // 16):
                        sl = pl.ds(j * 16, 16)
                        rows_v[r, sl] = jnp.maximum(
                            rows_v[r, sl] + qrows_v[r, sl], 0.0)

            pltpu.sync_copy(rows_v, acc.at[dst_v], add=True)

        @pl.loop(0, n_chunks // 2)
        def _(h):
            off0 = base + (2 * h) * ck
            cps_a = issue(off0, src_a, dst_a, rows_a, qrows_a,
                          sem_a0, sem_q0)
            cps_b = issue(off0 + ck, src_b, dst_b, rows_b, qrows_b,
                          sem_a1, sem_q1)
            finish(cps_a, dst_a, rows_a, qrows_a)
            finish(cps_b, dst_b, rows_b, qrows_b)

        if n_chunks % 2:
            off_l = base + (n_chunks - 1) * ck
            cps_l = issue(off_l, src_a, dst_a, rows_a, qrows_a,
                          sem_a0, sem_q0)
            finish(cps_l, dst_a, rows_a, qrows_a)

        plsc.subcore_barrier()
        pltpu.sync_copy(acc.at[pl.ds(r0, rows_sub)],
                        out_sum.at[cid, pl.ds(r0, rows_sub)])

    ins = [ta] + ([tq] if two else []) + [src, dst]
    fn = pl.kernel(body, out_type=out_type, mesh=_MESH, scratch_types=scratch,
                   name=name)
    return fn(*ins)


def _hist_call(dst, n_acc, *, name):
    e_pad = dst.shape[0]
    e_per_w = e_pad // NW
    n_chunks = e_per_w // K
    rows_sub = n_acc // NS

    def body(dst_ref, out_cnt, dst_a, dst_b, ones_v, cnt_acc, sem_a, sem_b):
        cid = lax.axis_index("c")
        sid = lax.axis_index("s")
        wid = sid * NC + cid
        r0 = sid * rows_sub

        @pl.loop(0, K)
        def _(r):
            for j in range(C // 16):
                ones_v[r, pl.ds(j * 16, 16)] = jnp.full((16,), 0.0,
                                                        jnp.float32)

        nfull, rem = rows_sub // K, rows_sub % K
        for k in range(nfull):
            pltpu.sync_copy(ones_v, cnt_acc.at[pl.ds(r0 + k * K, K)])
        if rem:
            pltpu.sync_copy(ones_v.at[pl.ds(0, rem)],
                            cnt_acc.at[pl.ds(r0 + nfull * K, rem)])

        @pl.loop(0, K)
        def _(r):
            for j in range(C // 16):
                ones_v[r, pl.ds(j * 16, 16)] = jnp.full((16,), 1.0,
                                                        jnp.float32)

        plsc.subcore_barrier()
        base = wid * e_per_w

        @pl.loop(0, n_chunks // 2)
        def _(h):
            off0 = base + (2 * h) * K
            cp_a = pltpu.async_copy(dst_ref.at[pl.ds(off0, K)], dst_a, sem_a)
            cp_b = pltpu.async_copy(dst_ref.at[pl.ds(off0 + K, K)], dst_b,
                                    sem_b)
            cp_a.wait()
            pltpu.sync_copy(ones_v, cnt_acc.at[dst_a], add=True)
            cp_b.wait()
            pltpu.sync_copy(ones_v, cnt_acc.at[dst_b], add=True)

        if n_chunks % 2:
            off_l = base + (n_chunks - 1) * K
            pltpu.sync_copy(dst_ref.at[pl.ds(off_l, K)], dst_a)
            pltpu.sync_copy(ones_v, cnt_acc.at[dst_a], add=True)

        plsc.subcore_barrier()
        pltpu.sync_copy(cnt_acc.at[pl.ds(r0, rows_sub)],
                        out_cnt.at[cid, pl.ds(r0, rows_sub)])

    fn = pl.kernel(body,
                   out_type=jax.ShapeDtypeStruct((NC, n_acc, C), jnp.float32),
                   mesh=_MESH,
                   scratch_types=[pltpu.VMEM((K,), jnp.int32),
                                  pltpu.VMEM((K,), jnp.int32),
                                  pltpu.VMEM((K, C), jnp.float32),
                                  pltpu.VMEM_SHARED((n_acc, C), jnp.float32),
                                  pltpu.SemaphoreType.DMA,
                                  pltpu.SemaphoreType.DMA],
                   name=name)
    return fn(dst)


def _sample_combine(hs0, hs1, hc0, hc1, idx, *, name):
    g = idx.shape[0]
    per_w = g // NW
    ck = next(c for c in range(min(per_w, K), 0, -8) if per_w % c == 0)
    n_chunks = per_w // ck

    def body(hs0_ref, hs1_ref, hc0_ref, hc1_ref, idx_ref, out_ref,
             idx_v, p0, p1, c0, c1, s0, s1, s2, s3):
        cid = lax.axis_index("c")
        sid = lax.axis_index("s")
        wid = sid * NC + cid
        base = wid * per_w

        @pl.loop(0, n_chunks)
        def _(ci):
            off = base + ci * ck
            pltpu.sync_copy(idx_ref.at[pl.ds(off, ck)], idx_v)
            cp0 = pltpu.async_copy(hs0_ref.at[idx_v], p0, s0)
            cp1 = pltpu.async_copy(hs1_ref.at[idx_v], p1, s1)
            cp2 = pltpu.async_copy(hc0_ref.at[idx_v], c0, s2)
            cp3 = pltpu.async_copy(hc1_ref.at[idx_v], c1, s3)
            cp0.wait()
            cp1.wait()
            cp2.wait()
            cp3.wait()

            @pl.loop(0, ck)
            def _(r):
                for j in range(C // 16):
                    sl = pl.ds(j * 16, 16)
                    d = jnp.maximum(c0[r, sl] + c1[r, sl], 1.0)
                    p0[r, sl] = (p0[r, sl] + p1[r, sl]) / d

            pltpu.sync_copy(p0, out_ref.at[pl.ds(off, ck)])

    fn = pl.kernel(body,
                   out_type=jax.ShapeDtypeStruct((g, C), jnp.float32),
                   mesh=_MESH,
                   scratch_types=[pltpu.VMEM((ck,), jnp.int32),
                                  pltpu.VMEM((ck, C), jnp.float32),
                                  pltpu.VMEM((ck, C), jnp.float32),
                                  pltpu.VMEM((ck, C), jnp.float32),
                                  pltpu.VMEM((ck, C), jnp.float32),
                                  pltpu.SemaphoreType.DMA,
                                  pltpu.SemaphoreType.DMA,
                                  pltpu.SemaphoreType.DMA,
                                  pltpu.SemaphoreType.DMA],
                   name=name)
    return fn(hs0, hs1, hc0, hc1, idx)


def _tc(fn, out_shape, *args, name):
    return pl.pallas_call(fn, out_shape=out_shape, name=name)(*args)


def _tc1(bxyz, feat, Wd, Wp, bd):
    def body(bxyz_ref, feat_ref, wd_ref, wp_ref, bd_ref, a_ref, q_ref):
        q = (bxyz_ref[:, 1:2] * wp_ref[0:1, :]
             + bxyz_ref[:, 2:3] * wp_ref[1:2, :]
             + bxyz_ref[:, 3:4] * wp_ref[2:3, :])
        a = jnp.dot(feat_ref[...], wd_ref[...], precision=_HI) + bd_ref[...] - q
        a_ref[...] = a
        q_ref[...] = q

    return _tc(body, [jax.ShapeDtypeStruct((N, C), jnp.float32),
                      jax.ShapeDtypeStruct((N, C), jnp.float32)],
               bxyz, feat, Wd, Wp, bd, name="tc1_aq")


def _combine(sums_ref, cnt_ref, n):
    s = sums_ref[0, :n, :] + sums_ref[1, :n, :]
    d = cnt_ref[0, :n, 0:1] + cnt_ref[1, :n, 0:1]
    return s / jnp.maximum(d, 1.0), d


def _tc3(S0, c2, F0p, Wf, Wfs, bf):
    def body(s_ref, c_ref, f0_ref, wf_ref, wfs_ref, bf_ref, f1_ref):
        A0, _ = _combine(s_ref, c_ref, N2)
        f0 = f0_ref[:N2, :]
        f1_ref[...] = jax.nn.relu(jnp.dot(A0, wf_ref[...], precision=_HI)
                                  + jnp.dot(f0, wfs_ref[...], precision=_HI)
                                  + bf_ref[...])

    return _tc(body, jax.ShapeDtypeStruct((N2, C), jnp.float32),
               S0, c2, F0p, Wf, Wfs, bf, name="tc3_flat")


def _tc4(S1, c2, Ws1, bs1):
    def body(s_ref, c_ref, w_ref, b_ref, a1_ref, s1_ref):
        A1, _ = _combine(s_ref, c_ref, N2)
        a1_ref[...] = A1
        s1_ref[...] = jax.nn.relu(jnp.dot(A1, w_ref[...], precision=_HI)
                                  + b_ref[...])

    return _tc(body, [jax.ShapeDtypeStruct((N2, C), jnp.float32),
                      jax.ShapeDtypeStruct((N2, C), jnp.float32)],
               S1, c2, Ws1, bs1, name="tc4_skip1")


def _tc5(S2, c2, F1, Ws2, bs2):
    def body(s_ref, c_ref, f1_ref, w_ref, b_ref, skip_ref):
        A2, _ = _combine(s_ref, c_ref, N2)
        s2 = jnp.dot(A2, w_ref[...], precision=_HI) + b_ref[...]
        skip_ref[...] = jax.nn.relu(s2 + f1_ref[...])

    return _tc(body, jax.ShapeDtypeStruct((N2, C), jnp.float32),
               S2, c2, F1, Ws2, bs2, name="tc5_skip2")


def _tc6(S3, c2, A1, F1, skip, Wm1, Wm2, bm, P1, P2, Wu):
    def body(s_ref, c_ref, a1_ref, f1_ref, sk_ref, wm1_ref, wm2_ref, bm_ref,
             p1_ref, p2_ref, wu_ref, g_ref):
        A3, _ = _combine(s_ref, c_ref, N2)
        merged = jax.nn.relu(jnp.dot(a1_ref[...], wm1_ref[...], precision=_HI)
                             + jnp.dot(A3, wm2_ref[...], precision=_HI)
                             + bm_ref[...])
        fr2 = (merged
               + jnp.dot(f1_ref[...], p1_ref[...], precision=_HI)
               + jnp.dot(sk_ref[...], p2_ref[...], precision=_HI))
        g_ref[...] = jnp.dot(fr2, wu_ref[...], precision=_HI)

    return _tc(body, jax.ShapeDtypeStruct((N2, C), jnp.float32),
               S3, c2, A1, F1, skip, Wm1, Wm2, bm, P1, P2, Wu,
               name="tc6_merge")


def _tc7(G, cu, bu):
    def body(g_ref, c_ref, b_ref, out_ref):
        avg, d = _combine(g_ref, c_ref, N)
        gate = jnp.where(d > 0.0, 1.0, 0.0)
        out_ref[...] = jax.nn.relu(avg + gate * b_ref[...])

    return _tc(body, jax.ShapeDtypeStruct((N, C), jnp.float32), G, cu, bu,
               name="tc7_up")


def kernel(point_bxyz, point_feat, edge_index, sample_idx, edge_index_down,
           up_src, up_dst, W_down, W_pos, b_down, W_flat, W_flat_self, b_flat,
           W_skip1, b_skip1, W_skip2, b_skip2, W_merge, b_merge, W_up, b_up):
    f32 = jnp.float32
    src0 = jnp.pad(edge_index[0], (0, E0P - E0))
    dst0 = jnp.pad(edge_index[1], (0, E0P - E0), constant_values=N)
    src2 = jnp.pad(edge_index_down[0], (0, E1P - E1))
    dst2 = jnp.pad(edge_index_down[1], (0, E1P - E1), constant_values=N2)
    usrc = jnp.pad(up_src, (0, EUP - EU))
    udst = jnp.pad(up_dst, (0, EUP - EU), constant_values=N)
    sidx = jnp.pad(sample_idx, (0, GPAD - N2))

    bd = b_down.reshape(1, C)
    bf = b_flat.reshape(1, C)
    bs1 = b_skip1.reshape(1, C)
    bs2 = b_skip2.reshape(1, C)
    bm = b_merge.reshape(1, C)
    bu = b_up.reshape(1, C)
    Wm1, Wm2 = W_merge[:C], W_merge[C:]
    P1 = jnp.asarray(_P1)
    P2 = jnp.asarray(_P2)

    a, q = _tc1(point_bxyz, point_feat, W_down, W_pos, bd)
    qp = jnp.pad(q, ((0, NACC_N - N), (0, 0)))
    hs = _seg_call(a, qp, src0, dst0, NACC_N, name="sc_down")
    hc = _hist_call(dst0, NACC_N, name="sc_down_deg")
    F0p = _sample_combine(hs[0], hs[1], hc[0], hc[1], sidx, name="sc_sample")
    S0 = _seg_call(F0p, None, src2, dst2, NACC_N2, name="sc_seg0")
    c2 = _hist_call(dst2, NACC_N2, name="sc_seg_deg")
    F1 = _tc3(S0, c2, F0p, W_flat, W_flat_self, bf)
    S1 = _seg_call(F1, None, src2, dst2, NACC_N2, name="sc_seg1")
    A1, s1 = _tc4(S1, c2, W_skip1, bs1)
    S2 = _seg_call(s1, None, src2, dst2, NACC_N2, name="sc_seg2")
    skip = _tc5(S2, c2, F1, W_skip2, bs2)
    S3 = _seg_call(skip, None, src2, dst2, NACC_N2, name="sc_seg3")
    g = _tc6(S3, c2, A1, F1, skip, Wm1, Wm2, bm, P1, P2, W_up)
    G = _seg_call(g, None, usrc, udst, NACC_N, name="sc_up")
    cu = _hist_call(udst, NACC_N, name="sc_up_deg")
    return _tc7(G, cu, bu)
